# baseline (speedup 1.0000x reference)
import jax
import jax.numpy as jnp
from jax.experimental import pallas as pl

_B = 512
_S = 1024
_R1 = 8192
_R2 = 24576
_NS = 256
_NM = 256
_NSMT = 4096
_LF = 1e-6
_NAL = 2.0
_EPS = 1e-30

_C1 = 1024
_C2 = 1024
_CS = 2048
_HI = (_R1 + _R2) // 128

_INTERPRET = False


def _sigmoid(x):
    return 1.0 / (1.0 + jnp.exp(-x))


def _med(t_col):
    Tg = 10.0 + 290.0 * _sigmoid(1e-3 * t_col)
    return jnp.log(Tg / 300.0), 1.0 / Tg


def _p1st_kernel(t_ref, y_ref, a_ref, b_ref, g_ref, r11_ref, p1_ref,
                 surf_ref, mant_ref, surfc_ref, smtr_ref, smtc_ref,
                 ra_ref, net_ref, ys_ref, ym_ref, cntr_ref, kmat_ref):
    i = pl.program_id(0)

    @pl.when(i == 0)
    def _init():
        iota_s = jax.lax.broadcasted_iota(jnp.int32, (_S, _NS), 0)
        cnt = jnp.sum((iota_s == surf_ref[...]).astype(jnp.float32),
                      axis=1, keepdims=True)
        cntm = jnp.sum((iota_s == mant_ref[...]).astype(jnp.float32),
                       axis=1, keepdims=True)
        iota_l = jax.lax.broadcasted_iota(jnp.int32, (_NS, _S), 1)
        cntr_ref[...] = jnp.sum((iota_l == surfc_ref[...]).astype(jnp.float32),
                                axis=0, keepdims=True)
        ys_ref[...] = jnp.dot(y_ref[...], cnt,
                              preferred_element_type=jnp.float32)
        ym_ref[...] = jnp.dot(y_ref[...], cntm,
                              preferred_element_type=jnp.float32)
        hi_row = smtr_ref[...] // 128
        lo_col = smtc_ref[...] % 128
        mh = (jax.lax.broadcasted_iota(jnp.int32, (_HI, _NSMT), 0)
              == hi_row).astype(jnp.float32)
        ml = (jax.lax.broadcasted_iota(jnp.int32, (_NSMT, 128), 1)
              == lo_col).astype(jnp.float32)
        kmat_ref[...] = jnp.dot(mh, ml, preferred_element_type=jnp.float32)
        net_ref[...] = jnp.zeros_like(net_ref)

    L, invT = _med(t_ref[...])
    c = a_ref[...] * jnp.exp(b_ref[...] * L - g_ref[...] * invT)
    iota_sub = jax.lax.broadcasted_iota(jnp.int32, (_S, _C1), 0)
    G = (iota_sub == r11_ref[...]).astype(jnp.float32)
    P = (iota_sub == p1_ref[...]).astype(jnp.float32)
    yA = jnp.dot(y_ref[...], G, preferred_element_type=jnp.float32)
    ra = c * yA
    ra_ref[...] = ra
    w = jnp.dot(cntr_ref[...], P - G, preferred_element_type=jnp.float32)
    net_ref[...] += jnp.sum(ra * w, axis=1, keepdims=True)


def _p2nd_kernel(t_ref, y_ref, a_ref, b_ref, g_ref, r12_ref, r22_ref, p2_ref,
                 cntr_ref, rb_ref, net_ref):
    i = pl.program_id(0)

    @pl.when(i == 0)
    def _init():
        net_ref[...] = jnp.zeros_like(net_ref)

    t = t_ref[...]
    L, invT = _med(t)
    den = jnp.exp(4.0 + 2.0 * jnp.tanh(5e-4 * t))
    c = a_ref[...] * jnp.exp(b_ref[...] * L - g_ref[...] * invT)
    iota_sub = jax.lax.broadcasted_iota(jnp.int32, (_S, _C2), 0)
    Ga = (iota_sub == r12_ref[...]).astype(jnp.float32)
    Gb = (iota_sub == r22_ref[...]).astype(jnp.float32)
    P = (iota_sub == p2_ref[...]).astype(jnp.float32)
    yB1 = jnp.dot(y_ref[...], Ga, preferred_element_type=jnp.float32)
    yB2 = jnp.dot(y_ref[...], Gb, preferred_element_type=jnp.float32)
    rb = c * yB1 * yB2 * den
    rb_ref[...] = rb
    w = jnp.dot(cntr_ref[...], P - Ga - Gb,
                preferred_element_type=jnp.float32)
    net_ref[...] += jnp.sum(rb * w, axis=1, keepdims=True)


def _scale(net1, net2, ys, ym):
    nl = _LF * (ys + ym)
    decay = jnp.minimum(_NAL / (nl + _EPS), 1.0)
    sc = decay * _sigmoid(net1 + net2)
    return jnp.log(sc)


def _s1st_kernel(ra_ref, p1_ref, r11_ref, k_ref, n1_ref, n2_ref,
                 ys_ref, ym_ref, dy_ref):
    i = pl.program_id(0)

    @pl.when(i == 0)
    def _init():
        dy_ref[...] = jnp.zeros_like(dy_ref)

    lsc = _scale(n1_ref[...], n2_ref[...], ys_ref[...], ym_ref[...])
    S1 = jnp.exp(k_ref[...] * lsc)
    rs = ra_ref[...] * S1
    iota_lane = jax.lax.broadcasted_iota(jnp.int32, (_CS, _S), 1)
    M = ((iota_lane == p1_ref[...]).astype(jnp.float32)
         - (iota_lane == r11_ref[...]).astype(jnp.float32))
    dy_ref[...] += jnp.dot(rs, M, preferred_element_type=jnp.float32)


def _s2nd_kernel(rb_ref, p2_ref, r12_ref, r22_ref, k_ref, n1_ref, n2_ref,
                 ys_ref, ym_ref, dy1_ref, dy_ref):
    i = pl.program_id(0)

    @pl.when(i == 0)
    def _init():
        dy_ref[...] = dy1_ref[...]

    lsc = _scale(n1_ref[...], n2_ref[...], ys_ref[...], ym_ref[...])
    S2 = jnp.exp(k_ref[...] * lsc)
    rs = rb_ref[...] * S2
    iota_lane = jax.lax.broadcasted_iota(jnp.int32, (_CS, _S), 1)
    M = ((iota_lane == p2_ref[...]).astype(jnp.float32)
         - (iota_lane == r12_ref[...]).astype(jnp.float32)
         - (iota_lane == r22_ref[...]).astype(jnp.float32))
    dy_ref[...] += jnp.dot(rs, M, preferred_element_type=jnp.float32)


def _row(x, n):
    return x.astype(jnp.int32).reshape(1, n)


def _col(x, n):
    return x.astype(jnp.int32).reshape(n, 1)


def kernel(t_in, y_in, alpha_1st, beta_1st, gamma_1st, alpha_2nd, beta_2nd,
           gamma_2nd, r1_1st, p_1st, r1_2nd, r2_2nd, p_2nd,
           inds_surf, inds_mant, inds_smt):
    f32 = jnp.float32
    t_col = t_in.astype(f32).reshape(_B, 1)
    y = y_in.astype(f32)
    a1 = alpha_1st.astype(f32).reshape(1, _R1)
    b1 = beta_1st.astype(f32).reshape(1, _R1)
    g1 = gamma_1st.astype(f32).reshape(1, _R1)
    a2 = alpha_2nd.astype(f32).reshape(1, _R2)
    b2 = beta_2nd.astype(f32).reshape(1, _R2)
    g2 = gamma_2nd.astype(f32).reshape(1, _R2)

    const = lambda *bs: pl.BlockSpec(bs, lambda i: (0,) * len(bs))
    rowblk = lambda c: pl.BlockSpec((1, c), lambda i: (0, i))
    colblk = lambda c: pl.BlockSpec((c, 1), lambda i: (i, 0))

    n1 = _R1 // _C1
    ra, net1, ysurf, ymant, cntr, kmat = pl.pallas_call(
        _p1st_kernel,
        grid=(n1,),
        in_specs=[
            const(_B, 1), const(_B, _S),
            rowblk(_C1), rowblk(_C1), rowblk(_C1),
            rowblk(_C1), rowblk(_C1),
            const(1, _NS), const(1, _NM), const(_NS, 1),
            const(1, _NSMT), const(_NSMT, 1),
        ],
        out_specs=[
            pl.BlockSpec((_B, _C1), lambda i: (0, i)),
            const(_B, 1), const(_B, 1), const(_B, 1),
            const(1, _S), const(_HI, 128),
        ],
        out_shape=[
            jax.ShapeDtypeStruct((_B, _R1), f32),
            jax.ShapeDtypeStruct((_B, 1), f32),
            jax.ShapeDtypeStruct((_B, 1), f32),
            jax.ShapeDtypeStruct((_B, 1), f32),
            jax.ShapeDtypeStruct((1, _S), f32),
            jax.ShapeDtypeStruct((_HI, 128), f32),
        ],
        interpret=_INTERPRET,
    )(t_col, y, a1, b1, g1, _row(r1_1st, _R1), _row(p_1st, _R1),
      _row(inds_surf, _NS), _row(inds_mant, _NM), _col(inds_surf, _NS),
      _row(inds_smt, _NSMT), _col(inds_smt, _NSMT))

    n2 = _R2 // _C2
    rb, net2 = pl.pallas_call(
        _p2nd_kernel,
        grid=(n2,),
        in_specs=[
            const(_B, 1), const(_B, _S),
            rowblk(_C2), rowblk(_C2), rowblk(_C2),
            rowblk(_C2), rowblk(_C2), rowblk(_C2),
            const(1, _S),
        ],
        out_specs=[
            pl.BlockSpec((_B, _C2), lambda i: (0, i)),
            const(_B, 1),
        ],
        out_shape=[
            jax.ShapeDtypeStruct((_B, _R2), f32),
            jax.ShapeDtypeStruct((_B, 1), f32),
        ],
        interpret=_INTERPRET,
    )(t_col, y, a2, b2, g2, _row(r1_2nd, _R2), _row(r2_2nd, _R2),
      _row(p_2nd, _R2), cntr)

    k_row = kmat.reshape(1, _R1 + _R2)
    k1 = k_row[:, :_R1]
    k2 = k_row[:, _R1:]

    dy1 = pl.pallas_call(
        _s1st_kernel,
        grid=(_R1 // _CS,),
        in_specs=[
            pl.BlockSpec((_B, _CS), lambda i: (0, i)),
            colblk(_CS), colblk(_CS), rowblk(_CS),
            const(_B, 1), const(_B, 1), const(_B, 1), const(_B, 1),
        ],
        out_specs=const(_B, _S),
        out_shape=jax.ShapeDtypeStruct((_B, _S), f32),
        interpret=_INTERPRET,
    )(ra, _col(p_1st, _R1), _col(r1_1st, _R1), k1, net1, net2, ysurf, ymant)

    dy = pl.pallas_call(
        _s2nd_kernel,
        grid=(_R2 // _CS,),
        in_specs=[
            pl.BlockSpec((_B, _CS), lambda i: (0, i)),
            colblk(_CS), colblk(_CS), colblk(_CS), rowblk(_CS),
            const(_B, 1), const(_B, 1), const(_B, 1), const(_B, 1),
            const(_B, _S),
        ],
        out_specs=const(_B, _S),
        out_shape=jax.ShapeDtypeStruct((_B, _S), f32),
        interpret=_INTERPRET,
    )(rb, _col(p_2nd, _R2), _col(r1_2nd, _R2), _col(r2_2nd, _R2), k2,
      net1, net2, ysurf, ymant, dy1)

    return dy


# speedup vs baseline: 1.0785x; 1.0785x over previous
import jax
import jax.numpy as jnp
from jax.experimental import pallas as pl

_B = 512
_S = 1024
_R1 = 8192
_R2 = 24576
_NS = 256
_NM = 256
_NSMT = 4096
_LF = 1e-6
_NAL = 2.0
_EPS = 1e-30

_C1 = 1024
_CS = 1024
_HI = (_R1 + _R2) // 128
_N1 = _R1 // _C1
_N2 = _R2 // _C1
_M1 = _R1 // _CS
_M2 = _R2 // _CS

_INTERPRET = False


def _sigmoid(x):
    return 1.0 / (1.0 + jnp.exp(-x))


def _med(t_col):
    Tg = 10.0 + 290.0 * _sigmoid(1e-3 * t_col)
    return jnp.log(Tg / 300.0), 1.0 / Tg


def _p_kernel(t_ref, y_ref, a1_ref, b1_ref, g1_ref, r11_ref, p1_ref,
              a2_ref, b2_ref, g2_ref, r12_ref, r22_ref, p2_ref,
              surf_ref, mant_ref, smtr_ref, smtc_ref,
              ra_ref, rb_ref, net_ref, ys_ref, ym_ref, kmat_ref):
    i = pl.program_id(0)

    @pl.when(i == 0)
    def _init():
        iota_s = jax.lax.broadcasted_iota(jnp.int32, (_S, _NS), 0)
        cnt = jnp.sum((iota_s == surf_ref[...]).astype(jnp.float32),
                      axis=1, keepdims=True)
        cntm = jnp.sum((iota_s == mant_ref[...]).astype(jnp.float32),
                       axis=1, keepdims=True)
        ys_ref[...] = jnp.dot(y_ref[...], cnt,
                              preferred_element_type=jnp.float32)
        ym_ref[...] = jnp.dot(y_ref[...], cntm,
                              preferred_element_type=jnp.float32)
        hi_row = smtr_ref[...] // 128
        lo_col = smtc_ref[...] % 128
        mh = (jax.lax.broadcasted_iota(jnp.int32, (_HI, _NSMT), 0)
              == hi_row).astype(jnp.float32)
        ml = (jax.lax.broadcasted_iota(jnp.int32, (_NSMT, 128), 1)
              == lo_col).astype(jnp.float32)
        kmat_ref[...] = jnp.dot(mh, ml, preferred_element_type=jnp.float32)
        net_ref[...] = jnp.zeros_like(net_ref)

    iota_cs = jax.lax.broadcasted_iota(jnp.int32, (_S, _NS), 0)
    cntc = jnp.sum((iota_cs == surf_ref[...]).astype(jnp.float32),
                   axis=1, keepdims=True)
    L, invT = _med(t_ref[...])
    iota_sub = jax.lax.broadcasted_iota(jnp.int32, (_S, _C1), 0)

    @pl.when(i < _N1)
    def _first():
        c = a1_ref[...] * jnp.exp(b1_ref[...] * L - g1_ref[...] * invT)
        G = (iota_sub == r11_ref[...]).astype(jnp.float32)
        P = (iota_sub == p1_ref[...]).astype(jnp.float32)
        yA = jnp.dot(y_ref[...], G, preferred_element_type=jnp.float32)
        ra = c * yA
        ra_ref[...] = ra
        w = jnp.sum((P - G) * cntc, axis=0, keepdims=True)
        net_ref[...] += jnp.sum(ra * w, axis=1, keepdims=True)

    @pl.when(i >= _N1)
    def _second():
        t = t_ref[...]
        den = jnp.exp(4.0 + 2.0 * jnp.tanh(5e-4 * t))
        c = a2_ref[...] * jnp.exp(b2_ref[...] * L - g2_ref[...] * invT)
        Ga = (iota_sub == r12_ref[...]).astype(jnp.float32)
        Gb = (iota_sub == r22_ref[...]).astype(jnp.float32)
        P = (iota_sub == p2_ref[...]).astype(jnp.float32)
        yB1 = jnp.dot(y_ref[...], Ga, preferred_element_type=jnp.float32)
        yB2 = jnp.dot(y_ref[...], Gb, preferred_element_type=jnp.float32)
        rb = c * yB1 * yB2 * den
        rb_ref[...] = rb
        w = jnp.sum((P - Ga - Gb) * cntc, axis=0, keepdims=True)
        net_ref[...] += jnp.sum(rb * w, axis=1, keepdims=True)


def _scale(net, ys, ym):
    nl = _LF * (ys + ym)
    decay = jnp.minimum(_NAL / (nl + _EPS), 1.0)
    sc = decay * _sigmoid(net)
    return jnp.log(sc)


def _s_kernel(ra_ref, rb_ref, p1_ref, r11_ref, k1_ref,
              p2_ref, r12_ref, r22_ref, k2_ref,
              net_ref, ys_ref, ym_ref, dy_ref):
    i = pl.program_id(0)

    @pl.when(i == 0)
    def _init():
        dy_ref[...] = jnp.zeros_like(dy_ref)

    lsc = _scale(net_ref[...], ys_ref[...], ym_ref[...])
    iota_lane = jax.lax.broadcasted_iota(jnp.int32, (_CS, _S), 1)

    @pl.when(i < _M1)
    def _first():
        rs = ra_ref[...] * jnp.exp(k1_ref[...] * lsc)
        M = ((iota_lane == p1_ref[...]).astype(jnp.float32)
             - (iota_lane == r11_ref[...]).astype(jnp.float32))
        dy_ref[...] += jnp.dot(rs, M, preferred_element_type=jnp.float32)

    @pl.when(i >= _M1)
    def _second():
        rs = rb_ref[...] * jnp.exp(k2_ref[...] * lsc)
        M = ((iota_lane == p2_ref[...]).astype(jnp.float32)
             - (iota_lane == r12_ref[...]).astype(jnp.float32)
             - (iota_lane == r22_ref[...]).astype(jnp.float32))
        dy_ref[...] += jnp.dot(rs, M, preferred_element_type=jnp.float32)


def _row(x, n):
    return x.astype(jnp.int32).reshape(1, n)


def _col(x, n):
    return x.astype(jnp.int32).reshape(n, 1)


def kernel(t_in, y_in, alpha_1st, beta_1st, gamma_1st, alpha_2nd, beta_2nd,
           gamma_2nd, r1_1st, p_1st, r1_2nd, r2_2nd, p_2nd,
           inds_surf, inds_mant, inds_smt):
    f32 = jnp.float32
    t_col = t_in.astype(f32).reshape(_B, 1)
    y = y_in.astype(f32)
    a1 = alpha_1st.astype(f32).reshape(1, _R1)
    b1 = beta_1st.astype(f32).reshape(1, _R1)
    g1 = gamma_1st.astype(f32).reshape(1, _R1)
    a2 = alpha_2nd.astype(f32).reshape(1, _R2)
    b2 = beta_2nd.astype(f32).reshape(1, _R2)
    g2 = gamma_2nd.astype(f32).reshape(1, _R2)

    const = lambda *bs: pl.BlockSpec(bs, lambda i: (0,) * len(bs))

    c1a = lambda c: pl.BlockSpec((1, c), lambda i: (0, jnp.minimum(i, _N1 - 1)))
    c1b = lambda c: pl.BlockSpec(
        (1, c), lambda i: (0, jnp.maximum(i - _N1, 0)))

    ra, rb, net, ysurf, ymant, kmat = pl.pallas_call(
        _p_kernel,
        grid=(_N1 + _N2,),
        in_specs=[
            const(_B, 1), const(_B, _S),
            c1a(_C1), c1a(_C1), c1a(_C1), c1a(_C1), c1a(_C1),
            c1b(_C1), c1b(_C1), c1b(_C1), c1b(_C1), c1b(_C1), c1b(_C1),
            const(1, _NS), const(1, _NM),
            const(1, _NSMT), const(_NSMT, 1),
        ],
        out_specs=[
            pl.BlockSpec((_B, _C1), lambda i: (0, jnp.minimum(i, _N1 - 1))),
            pl.BlockSpec((_B, _C1), lambda i: (0, jnp.maximum(i - _N1, 0))),
            const(_B, 1), const(_B, 1), const(_B, 1),
            const(_HI, 128),
        ],
        out_shape=[
            jax.ShapeDtypeStruct((_B, _R1), f32),
            jax.ShapeDtypeStruct((_B, _R2), f32),
            jax.ShapeDtypeStruct((_B, 1), f32),
            jax.ShapeDtypeStruct((_B, 1), f32),
            jax.ShapeDtypeStruct((_B, 1), f32),
            jax.ShapeDtypeStruct((_HI, 128), f32),
        ],
        interpret=_INTERPRET,
    )(t_col, y,
      a1, b1, g1, _row(r1_1st, _R1), _row(p_1st, _R1),
      a2, b2, g2, _row(r1_2nd, _R2), _row(r2_2nd, _R2), _row(p_2nd, _R2),
      _row(inds_surf, _NS), _row(inds_mant, _NM),
      _row(inds_smt, _NSMT), _col(inds_smt, _NSMT))

    k_row = kmat.reshape(1, _R1 + _R2)
    k1 = k_row[:, :_R1]
    k2 = k_row[:, _R1:]

    csa = lambda c: pl.BlockSpec((c, 1), lambda i: (jnp.minimum(i, _M1 - 1), 0))
    csb = lambda c: pl.BlockSpec(
        (c, 1), lambda i: (jnp.maximum(i - _M1, 0), 0))
    csar = lambda c: pl.BlockSpec(
        (1, c), lambda i: (0, jnp.minimum(i, _M1 - 1)))
    csbr = lambda c: pl.BlockSpec(
        (1, c), lambda i: (0, jnp.maximum(i - _M1, 0)))

    dy = pl.pallas_call(
        _s_kernel,
        grid=(_M1 + _M2,),
        in_specs=[
            pl.BlockSpec((_B, _CS), lambda i: (0, jnp.minimum(i, _M1 - 1))),
            pl.BlockSpec((_B, _CS), lambda i: (0, jnp.maximum(i - _M1, 0))),
            csa(_CS), csa(_CS), csar(_CS),
            csb(_CS), csb(_CS), csb(_CS), csbr(_CS),
            const(_B, 1), const(_B, 1), const(_B, 1),
        ],
        out_specs=const(_B, _S),
        out_shape=jax.ShapeDtypeStruct((_B, _S), f32),
        interpret=_INTERPRET,
    )(ra, rb,
      _col(p_1st, _R1), _col(r1_1st, _R1), k1,
      _col(p_2nd, _R2), _col(r1_2nd, _R2), _col(r2_2nd, _R2), k2,
      net, ysurf, ymant)

    return dy
